# baseline (speedup 1.0000x reference)
import functools

import jax
import jax.numpy as jnp
from jax import lax
from jax.experimental import pallas as pl
from jax.experimental.pallas import tpu as pltpu
from jax.experimental.pallas import tpu_sc as plsc

NUM_FIELDS = 26
VOCAB = 100000
EMBED_DIM = 32
BATCH = 16384

_INFO = plsc.get_sparse_core_info()
_NC = _INFO.num_cores
_NS = _INFO.num_subcores
_NW = _NC * _NS
_L = 16

_VB = 12800
_NVB = -(-VOCAB // _VB)
_LINES_F = _NVB * (_VB // 4)

_CHUNK = 16
_ROWS_PER_W = BATCH // _NW
_CHUNKS_PER_W = _ROWS_PER_W // _CHUNK


def _tr_body(i_ref, o_ref):
    blk = i_ref[0]
    r = blk.reshape(128, _VB // 4)
    r3 = r.reshape(EMBED_DIM, 4, _VB // 4)
    zu = r3[:, 0:2, :].reshape(2 * EMBED_DIM, _VB // 4)
    zl = r3[:, 2:4, :].reshape(2 * EMBED_DIM, _VB // 4)
    o_ref[0] = jnp.concatenate([zu, zl], axis=0).T


def _relayout(w_t):
    return pl.pallas_call(
        _tr_body,
        grid=(NUM_FIELDS, _NVB),
        in_specs=[pl.BlockSpec((1, EMBED_DIM, _VB), lambda f, v: (f, 0, v))],
        out_specs=pl.BlockSpec((1, _VB // 4, 128), lambda f, v: (f, v, 0)),
        out_shape=jax.ShapeDtypeStruct((NUM_FIELDS, _LINES_F, 128), jnp.float32),
    )(w_t)


def _emb_body(lines_hbm, pos_hbm, w_hbm, out_hbm,
              lines_v, pos_v, rows_v, acc_v, sem0, sem1, osem):
    wid = lax.axis_index("s") * _NC + lax.axis_index("c")
    base = wid * _ROWS_PER_W
    sems = (sem0, sem1)
    iota = lax.iota(jnp.int32, _L)

    def fire(c, s):
        b0 = base + c * _CHUNK
        pltpu.sync_copy(lines_hbm.at[:, pl.ds(b0, _CHUNK)], lines_v.at[s])
        pltpu.sync_copy(pos_hbm.at[:, pl.ds(b0, _CHUNK)], pos_v.at[s])
        for f in range(NUM_FIELDS):
            pltpu.async_copy(w_hbm.at[lines_v.at[s, f]], rows_v.at[s, f],
                             sems[s])

    def drain(s):
        for f in range(NUM_FIELDS):
            pltpu.make_async_copy(w_hbm.at[lines_v.at[s, f]],
                                  rows_v.at[s, f], sems[s]).wait()

    def wait_out(s):
        pltpu.make_async_copy(
            acc_v.at[s], out_hbm.at[pl.ds(base, _CHUNK)], osem).wait()

    def reduce_chunk(s, c):
        buf = rows_v.at[s]
        pos = [pos_v[s, f, :] for f in range(NUM_FIELDS)]

        def dbody(d, cols):
            vec = plsc.load_gather(
                buf, [jnp.full((_L,), 0, jnp.int32), iota, cols[0]])
            for f in range(1, NUM_FIELDS):
                vec = vec + plsc.load_gather(
                    buf, [jnp.full((_L,), f, jnp.int32), iota, cols[f]])
            plsc.store_scatter(
                acc_v.at[s], [iota, jnp.full((_L,), 0, jnp.int32) + d], vec)
            return [cf + 2 for cf in cols]
        lax.fori_loop(0, EMBED_DIM, dbody, pos)
        pltpu.async_copy(
            acc_v.at[s], out_hbm.at[pl.ds(base + c * _CHUNK, _CHUNK)], osem)

    fire(0, 0)

    def ring(g, carry):
        c0 = 2 * g
        fire(c0 + 1, 1)

        drain(0)

        @pl.when(g > 0)
        def _():
            wait_out(0)
        reduce_chunk(0, c0)

        @pl.when(g < _CHUNKS_PER_W // 2 - 1)
        def _():
            fire(c0 + 2, 0)

        drain(1)

        @pl.when(g > 0)
        def _():
            wait_out(1)
        reduce_chunk(1, c0 + 1)
        return carry

    lax.fori_loop(0, _CHUNKS_PER_W // 2, ring, 0)
    wait_out(0)
    wait_out(1)


@functools.partial(jax.jit, static_argnames=())
def _emb(lines, pos, w_lines_flat):
    mesh = plsc.VectorSubcoreMesh(core_axis_name="c", subcore_axis_name="s")
    run = pl.kernel(
        _emb_body,
        out_type=jax.ShapeDtypeStruct((BATCH, EMBED_DIM), jnp.float32),
        mesh=mesh,
        compiler_params=pltpu.CompilerParams(
            use_tc_tiling_on_sc=False, needs_layout_passes=False),
        scratch_types=[
            pltpu.VMEM((2, NUM_FIELDS, _CHUNK), jnp.int32),
            pltpu.VMEM((2, NUM_FIELDS, _CHUNK), jnp.int32),
            pltpu.VMEM((2, NUM_FIELDS, _CHUNK, 64), jnp.float32),
            pltpu.VMEM((2, _CHUNK, EMBED_DIM), jnp.float32),
            pltpu.SemaphoreType.DMA,
            pltpu.SemaphoreType.DMA,
            pltpu.SemaphoreType.DMA,
        ],
    )
    return run(lines, pos, w_lines_flat)


def kernel(x, W):
    v = x.astype(jnp.int32)
    vb, vr = v // _VB, v % _VB
    line_f = vb * (_VB // 4) + vr % (_VB // 4)
    m = vr // (_VB // 4)
    offs = (jnp.arange(NUM_FIELDS, dtype=jnp.int32) * _LINES_F)[None, :]
    lines = ((line_f + offs) * 2 + m // 2).T
    pos = (m % 2).T

    w_lines = _relayout(jnp.swapaxes(W, 1, 2))
    w_flat = w_lines.reshape(NUM_FIELDS * _LINES_F * 2, 64)
    return _emb(lines, pos, w_flat)


# speedup vs baseline: 1.2842x; 1.2842x over previous
import functools

import jax
import jax.numpy as jnp
from jax import lax
from jax.experimental import pallas as pl
from jax.experimental.pallas import tpu as pltpu
from jax.experimental.pallas import tpu_sc as plsc

NUM_FIELDS = 26
VOCAB = 100000
EMBED_DIM = 32
BATCH = 16384

_INFO = plsc.get_sparse_core_info()
_NC = _INFO.num_cores
_NS = _INFO.num_subcores
_NW = _NC * _NS
_L = 16

_VB = 12800
_NVB = -(-VOCAB // _VB)
_LINES_F = _NVB * (_VB // 4)

_CHUNK = 16
_ROWS_PER_W = BATCH // _NW
_CHUNKS_PER_W = _ROWS_PER_W // _CHUNK


def _tr_body(i_ref, o_ref):
    blk = i_ref[0]
    r3 = blk.reshape(128, _VB // 4).reshape(EMBED_DIM, 4, _VB // 4)
    z = jnp.concatenate([r3[:, k, :] for k in range(4)], axis=0)
    o_ref[0] = z.T


def _relayout(w_t):
    return pl.pallas_call(
        _tr_body,
        grid=(NUM_FIELDS, _NVB),
        in_specs=[pl.BlockSpec((1, EMBED_DIM, _VB), lambda f, v: (f, 0, v))],
        out_specs=pl.BlockSpec((1, _VB // 4, 128), lambda f, v: (f, v, 0)),
        out_shape=jax.ShapeDtypeStruct((NUM_FIELDS, _LINES_F, 128), jnp.float32),
    )(w_t)


_C2 = 64
_NCH = _ROWS_PER_W // _C2
_VECS = EMBED_DIM // _L
_RUN = 2


def _reduce_chunk(buf, acc_v):
    def body(i, carry):
        for j in range(_RUN):
            r = i * _RUN + j
            for c in range(_VECS):
                sl = pl.ds(c * _L, _L)
                v = buf[0, r, sl]
                for f in range(1, NUM_FIELDS):
                    v = v + buf[f, r, sl]
                acc_v[r, sl] = v
        return carry
    lax.fori_loop(0, _C2 // _RUN, body, 0)


def _emb_body(idx_hbm, w_hbm, out_hbm, idx_v, rows_v, acc_v, sem0, sem1):
    wid = lax.axis_index("s") * _NC + lax.axis_index("c")
    base = wid * _ROWS_PER_W
    sems = (sem0, sem1)

    def fire(c, s):
        row0 = base + c * _C2
        pltpu.sync_copy(idx_hbm.at[:, pl.ds(row0, _C2)], idx_v.at[s])
        return [
            pltpu.async_copy(w_hbm.at[idx_v.at[s, f]], rows_v.at[s, f],
                             sems[s])
            for f in range(NUM_FIELDS)
        ]

    handles = [None, None]
    handles[0] = fire(0, 0)
    for c in range(1, _NCH + 1):
        s = c % 2
        if c < _NCH:
            handles[s] = fire(c, s)
        p = (c - 1) % 2
        for h in handles[p]:
            h.wait()
        _reduce_chunk(rows_v.at[p], acc_v)
        pltpu.sync_copy(acc_v, out_hbm.at[pl.ds(base + (c - 1) * _C2, _C2)])


@functools.partial(jax.jit, static_argnames=())
def _emb(idx, w_flat):
    mesh = plsc.VectorSubcoreMesh(core_axis_name="c", subcore_axis_name="s")
    run = pl.kernel(
        _emb_body,
        out_type=jax.ShapeDtypeStruct((BATCH, EMBED_DIM), jnp.float32),
        mesh=mesh,
        compiler_params=pltpu.CompilerParams(use_tc_tiling_on_sc=False),
        scratch_types=[
            pltpu.VMEM((2, NUM_FIELDS, _C2), jnp.int32),
            pltpu.VMEM((2, NUM_FIELDS, _C2, EMBED_DIM), jnp.float32),
            pltpu.VMEM((_C2, EMBED_DIM), jnp.float32),
            pltpu.SemaphoreType.DMA,
            pltpu.SemaphoreType.DMA,
        ],
    )
    return run(idx, w_flat)


def kernel(x, W):
    v = x.astype(jnp.int32)
    vb, vr = v // _VB, v % _VB
    line_f = vb * (_VB // 4) + vr % (_VB // 4)
    m = vr // (_VB // 4)
    offs = (jnp.arange(NUM_FIELDS, dtype=jnp.int32) * _LINES_F)[None, :]
    rows = ((line_f + offs) * 4 + m).T

    w_lines = _relayout(jnp.swapaxes(W, 1, 2))
    w_flat = w_lines.reshape(NUM_FIELDS * _LINES_F * 4, EMBED_DIM)
    return _emb(rows, w_flat)


# speedup vs baseline: 1.4657x; 1.1414x over previous
import functools

import jax
import jax.numpy as jnp
from jax import lax
from jax.experimental import pallas as pl
from jax.experimental.pallas import tpu as pltpu
from jax.experimental.pallas import tpu_sc as plsc

NUM_FIELDS = 26
VOCAB = 100000
EMBED_DIM = 32
BATCH = 16384

_INFO = plsc.get_sparse_core_info()
_NC = _INFO.num_cores
_NS = _INFO.num_subcores
_NW = _NC * _NS
_L = 16

_VB = 25600
_NVB = -(-VOCAB // _VB)
_LINES_F = _NVB * (_VB // 4)

_CHUNK = 16
_ROWS_PER_W = BATCH // _NW
_CHUNKS_PER_W = _ROWS_PER_W // _CHUNK


def _tr_body(i_ref, o_ref):
    blk = i_ref[0]
    r3 = blk.reshape(128, _VB // 4).reshape(EMBED_DIM, 4, _VB // 4)
    z = jnp.concatenate([r3[:, k, :] for k in range(4)], axis=0)
    o_ref[0] = z.T


def _relayout(w_t):
    return pl.pallas_call(
        _tr_body,
        grid=(NUM_FIELDS, _NVB),
        in_specs=[pl.BlockSpec((1, EMBED_DIM, _VB), lambda f, v: (f, 0, v))],
        out_specs=pl.BlockSpec((1, _VB // 4, 128), lambda f, v: (f, v, 0)),
        out_shape=jax.ShapeDtypeStruct((NUM_FIELDS, _LINES_F, 128), jnp.float32),
    )(w_t)


_C2 = 64
_NCH = _ROWS_PER_W // _C2
_VECS = EMBED_DIM // _L
_RUN = 2


def _reduce_chunk(buf, acc_v):
    def body(i, carry):
        for j in range(_RUN):
            r = i * _RUN + j
            for c in range(_VECS):
                sl = pl.ds(c * _L, _L)
                v = buf[0, r, sl]
                for f in range(1, NUM_FIELDS):
                    v = v + buf[f, r, sl]
                acc_v[r, sl] = v
        return carry
    lax.fori_loop(0, _C2 // _RUN, body, 0)


def _emb_body(idx_hbm, w_hbm, out_hbm, idx_v, rows_v, acc_v, sem0, sem1):
    wid = lax.axis_index("s") * _NC + lax.axis_index("c")
    base = wid * _ROWS_PER_W
    sems = (sem0, sem1)

    def fire(c, s):
        row0 = base + c * _C2
        pltpu.sync_copy(idx_hbm.at[:, pl.ds(row0, _C2)], idx_v.at[s])
        return [
            pltpu.async_copy(w_hbm.at[idx_v.at[s, f]], rows_v.at[s, f],
                             sems[s])
            for f in range(NUM_FIELDS)
        ]

    handles = [None, None]
    handles[0] = fire(0, 0)
    for c in range(1, _NCH + 1):
        s = c % 2
        if c < _NCH:
            handles[s] = fire(c, s)
        p = (c - 1) % 2
        for h in handles[p]:
            h.wait()
        _reduce_chunk(rows_v.at[p], acc_v)
        pltpu.sync_copy(acc_v, out_hbm.at[pl.ds(base + (c - 1) * _C2, _C2)])


@functools.partial(jax.jit, static_argnames=())
def _emb(idx, w_flat):
    mesh = plsc.VectorSubcoreMesh(core_axis_name="c", subcore_axis_name="s")
    run = pl.kernel(
        _emb_body,
        out_type=jax.ShapeDtypeStruct((BATCH, EMBED_DIM), jnp.float32),
        mesh=mesh,
        compiler_params=pltpu.CompilerParams(use_tc_tiling_on_sc=False),
        scratch_types=[
            pltpu.VMEM((2, NUM_FIELDS, _C2), jnp.int32),
            pltpu.VMEM((2, NUM_FIELDS, _C2, EMBED_DIM), jnp.float32),
            pltpu.VMEM((_C2, EMBED_DIM), jnp.float32),
            pltpu.SemaphoreType.DMA,
            pltpu.SemaphoreType.DMA,
        ],
    )
    return run(idx, w_flat)


def kernel(x, W):
    v = x.astype(jnp.int32)
    vb, vr = v // _VB, v % _VB
    line_f = vb * (_VB // 4) + vr % (_VB // 4)
    m = vr // (_VB // 4)
    offs = (jnp.arange(NUM_FIELDS, dtype=jnp.int32) * _LINES_F)[None, :]
    rows = ((line_f + offs) * 4 + m).T

    w_lines = _relayout(jnp.swapaxes(W, 1, 2))
    w_flat = w_lines.reshape(NUM_FIELDS * _LINES_F * 4, EMBED_DIM)
    return _emb(rows, w_flat)


# speedup vs baseline: 1.6080x; 1.0971x over previous
import functools

import jax
import jax.numpy as jnp
from jax import lax
from jax.experimental import pallas as pl
from jax.experimental.pallas import tpu as pltpu
from jax.experimental.pallas import tpu_sc as plsc

NUM_FIELDS = 26
VOCAB = 100000
EMBED_DIM = 32
BATCH = 16384

_INFO = plsc.get_sparse_core_info()
_NC = _INFO.num_cores
_NS = _INFO.num_subcores
_NW = _NC * _NS
_L = 16

_VB = 51200
_NVB = -(-VOCAB // _VB)
_LINES_F = _NVB * (_VB // 4)

_CHUNK = 16
_ROWS_PER_W = BATCH // _NW
_CHUNKS_PER_W = _ROWS_PER_W // _CHUNK


def _tr_body(i_ref, o_ref):
    blk = i_ref[0]
    r3 = blk.reshape(128, _VB // 4).reshape(EMBED_DIM, 4, _VB // 4)
    z = jnp.concatenate([r3[:, k, :] for k in range(4)], axis=0)
    o_ref[0] = z.T


def _relayout(w_t):
    return pl.pallas_call(
        _tr_body,
        grid=(NUM_FIELDS, _NVB),
        in_specs=[pl.BlockSpec((1, EMBED_DIM, _VB), lambda f, v: (f, 0, v))],
        out_specs=pl.BlockSpec((1, _VB // 4, 128), lambda f, v: (f, v, 0)),
        out_shape=jax.ShapeDtypeStruct((NUM_FIELDS, _LINES_F, 128), jnp.float32),
    )(w_t)


_C2 = 64
_NCH = _ROWS_PER_W // _C2
_VECS = EMBED_DIM // _L
_RUN = 2


def _reduce_chunk(buf, acc_v):
    def body(i, carry):
        for j in range(_RUN):
            r = i * _RUN + j
            for c in range(_VECS):
                sl = pl.ds(c * _L, _L)
                v = buf[0, r, sl]
                for f in range(1, NUM_FIELDS):
                    v = v + buf[f, r, sl]
                acc_v[r, sl] = v
        return carry
    lax.fori_loop(0, _C2 // _RUN, body, 0)


def _emb_body(idx_hbm, w_hbm, out_hbm, idx_v, rows_v, acc_v, sem0, sem1):
    wid = lax.axis_index("s") * _NC + lax.axis_index("c")
    base = wid * _ROWS_PER_W
    sems = (sem0, sem1)

    def fire(c, s):
        row0 = base + c * _C2
        pltpu.sync_copy(idx_hbm.at[:, pl.ds(row0, _C2)], idx_v.at[s])
        return [
            pltpu.async_copy(w_hbm.at[idx_v.at[s, f]], rows_v.at[s, f],
                             sems[s])
            for f in range(NUM_FIELDS)
        ]

    handles = [None, None]
    handles[0] = fire(0, 0)
    for c in range(1, _NCH + 1):
        s = c % 2
        if c < _NCH:
            handles[s] = fire(c, s)
        p = (c - 1) % 2
        for h in handles[p]:
            h.wait()
        _reduce_chunk(rows_v.at[p], acc_v)
        pltpu.sync_copy(acc_v, out_hbm.at[pl.ds(base + (c - 1) * _C2, _C2)])


@functools.partial(jax.jit, static_argnames=())
def _emb(idx, w_flat):
    mesh = plsc.VectorSubcoreMesh(core_axis_name="c", subcore_axis_name="s")
    run = pl.kernel(
        _emb_body,
        out_type=jax.ShapeDtypeStruct((BATCH, EMBED_DIM), jnp.float32),
        mesh=mesh,
        compiler_params=pltpu.CompilerParams(use_tc_tiling_on_sc=False),
        scratch_types=[
            pltpu.VMEM((2, NUM_FIELDS, _C2), jnp.int32),
            pltpu.VMEM((2, NUM_FIELDS, _C2, EMBED_DIM), jnp.float32),
            pltpu.VMEM((_C2, EMBED_DIM), jnp.float32),
            pltpu.SemaphoreType.DMA,
            pltpu.SemaphoreType.DMA,
        ],
    )
    return run(idx, w_flat)


def kernel(x, W):
    v = x.astype(jnp.int32)
    vb, vr = v // _VB, v % _VB
    line_f = vb * (_VB // 4) + vr % (_VB // 4)
    m = vr // (_VB // 4)
    offs = (jnp.arange(NUM_FIELDS, dtype=jnp.int32) * _LINES_F)[None, :]
    rows = ((line_f + offs) * 4 + m).T

    w_lines = _relayout(jnp.swapaxes(W, 1, 2))
    w_flat = w_lines.reshape(NUM_FIELDS * _LINES_F * 4, EMBED_DIM)
    return _emb(rows, w_flat)
